# Initial kernel scaffold; baseline (speedup 1.0000x reference)
import jax
import jax.numpy as jnp
from jax.experimental import pallas as pl


def kernel(rays_o, rays_d, occ_grid):
    raise NotImplementedError("write your pallas kernel here")


# speedup vs baseline: 284.9411x; 284.9411x over previous
import functools

import jax
import jax.numpy as jnp
from jax import lax
from jax.experimental import pallas as pl
from jax.experimental.pallas import tpu as pltpu
from jax.experimental.pallas import tpu_sc as plsc

_N_RAYS = 65536
_RES = 128
_NSAMP = 128
_NEAR = 0.1
_FAR = 3.0
_OCC_THRES = 0.01
_TAB = _RES * _RES * _RES
_PADN = 65536

_RB = 1024
_ROWS = 8


def _idx_body(t_ref, ox_ref, oy_ref, oz_ref, dx_ref, dy_ref, dz_ref, idx_ref):
    b = pl.program_id(0)
    t = t_ref[...]
    dxv, dyv, dzv = dx_ref[...], dy_ref[...], dz_ref[...]
    norm = jnp.sqrt(dxv * dxv + dyv * dyv + dzv * dzv) + 1e-8
    inv = 64.0 / norm
    ax = ox_ref[...] * 64.0 + 64.0
    ay = oy_ref[...] * 64.0 + 64.0
    az = oz_ref[...] * 64.0 + 64.0
    bx, by, bz = dxv * inv, dyv * inv, dzv * inv

    ux = ax + bx * t
    uy = ay + by * t
    uz = az + bz * t
    ix = jnp.clip(ux.astype(jnp.int32), 0, _RES - 1)
    iy = jnp.clip(uy.astype(jnp.int32), 0, _RES - 1)
    iz = jnp.clip(uz.astype(jnp.int32), 0, _RES - 1)
    inside = (
        (ux >= 0.0) & (ux < 128.0)
        & (uy >= 0.0) & (uy < 128.0)
        & (uz >= 0.0) & (uz < 128.0)
    )
    flat = (ix << 14) + (iy << 7) + iz
    r = lax.broadcasted_iota(jnp.int32, (_NSAMP, _RB), 1) + b * _RB
    s = lax.broadcasted_iota(jnp.int32, (_NSAMP, _RB), 0)
    pad = _TAB + (((r << 7) + s) & (_PADN - 1))
    idx_ref[...] = jnp.where(inside, flat, pad)


def _compute_idx(t_mid, ox, oy, oz, dx, dy, dz):
    nblk = _N_RAYS // _RB
    ray_spec = pl.BlockSpec((1, _RB), lambda b: (0, b))
    return pl.pallas_call(
        _idx_body,
        grid=(nblk,),
        in_specs=[pl.BlockSpec((_NSAMP, 1), lambda b: (0, 0))] + [ray_spec] * 6,
        out_specs=pl.BlockSpec((_NSAMP, _RB), lambda b: (0, b)),
        out_shape=jax.ShapeDtypeStruct((_NSAMP, _N_RAYS), jnp.int32),
    )(t_mid.reshape(_NSAMP, 1), ox, oy, oz, dx, dy, dz)


def _sc_gather(table, idx_flat):
    nw = 32
    b_total = idx_flat.shape[0]
    b_per_w = b_total // nw
    ch = 16384
    nch = b_per_w // ch
    mesh = plsc.VectorSubcoreMesh(core_axis_name="c", subcore_axis_name="s")

    @functools.partial(
        pl.kernel,
        out_type=jax.ShapeDtypeStruct((b_total,), jnp.float32),
        mesh=mesh,
        scratch_types=[
            pltpu.VMEM((ch,), jnp.int32),
            pltpu.VMEM((ch,), jnp.float32),
            pltpu.SemaphoreType.DMA,
        ],
    )
    def gather_k(tab_hbm, idx_hbm, out_hbm, idx_v, occ_v, sem):
        wid = lax.axis_index("s") * 2 + lax.axis_index("c")
        base = wid * b_per_w

        def body(c, carry):
            off = base + c * ch
            pltpu.sync_copy(idx_hbm.at[pl.ds(off, ch)], idx_v)
            pltpu.async_copy(tab_hbm.at[idx_v], occ_v, sem).wait()
            pltpu.sync_copy(occ_v, out_hbm.at[pl.ds(off, ch)])
            return carry

        lax.fori_loop(0, nch, body, 0)

    return gather_k(table, idx_flat)


def _comp_body(t_ref, dt_ref, occ_ref, ox_ref, oy_ref, oz_ref,
               dx_ref, dy_ref, dz_ref, cx_ref, cy_ref, cz_ref):
    dxv, dyv, dzv = dx_ref[...], dy_ref[...], dz_ref[...]
    norm = jnp.sqrt(dxv * dxv + dyv * dyv + dzv * dzv) + 1e-8
    inv = 1.0 / norm
    ndx, ndy, ndz = dxv * inv, dyv * inv, dzv * inv

    shape = (_ROWS, 128)

    def body(sidx, carry):
        trans, wsum, tsum = carry
        occ_s = occ_ref[sidx]
        sp = jnp.log1p(jnp.exp(occ_s))
        sigma = jnp.where(occ_s > _OCC_THRES, sp, 0.0)
        one_m_alpha = jnp.exp(-sigma * dt_ref[sidx])
        alpha = 1.0 - one_m_alpha
        w = alpha * trans
        wsum = wsum + w
        tsum = tsum + w * t_ref[sidx]
        trans = trans * (one_m_alpha + 1e-10)
        return trans, wsum, tsum

    init = (jnp.ones(shape, jnp.float32),
            jnp.zeros(shape, jnp.float32),
            jnp.zeros(shape, jnp.float32))
    _, wsum, tsum = lax.fori_loop(0, _NSAMP, body, init)

    cx_ref[...] = ox_ref[...] * wsum + ndx * tsum
    cy_ref[...] = oy_ref[...] * wsum + ndy * tsum
    cz_ref[...] = oz_ref[...] * wsum + ndz * tsum


def _composite(t_mid, dt, occ, ox, oy, oz, dx, dy, dz):
    nrows = _N_RAYS // 128
    nblk = nrows // _ROWS
    smem_spec = pl.BlockSpec(memory_space=pltpu.SMEM)
    ray_spec = pl.BlockSpec((_ROWS, 128), lambda b: (b, 0))
    out_sds = jax.ShapeDtypeStruct((nrows, 128), jnp.float32)
    return pl.pallas_call(
        _comp_body,
        grid=(nblk,),
        in_specs=[smem_spec, smem_spec,
                  pl.BlockSpec((_NSAMP, _ROWS, 128), lambda b: (0, b, 0))]
                 + [ray_spec] * 6,
        out_specs=[ray_spec] * 3,
        out_shape=[out_sds, out_sds, out_sds],
    )(t_mid, dt, occ, ox, oy, oz, dx, dy, dz)


def kernel(rays_o, rays_d, occ_grid):
    f32 = jnp.float32
    t_edges = jnp.linspace(_NEAR, _FAR, _NSAMP + 1, dtype=f32)
    t_mid = 0.5 * (t_edges[:-1] + t_edges[1:])
    dt = t_edges[1:] - t_edges[:-1]

    ox = rays_o[:, 0].reshape(1, _N_RAYS)
    oy = rays_o[:, 1].reshape(1, _N_RAYS)
    oz = rays_o[:, 2].reshape(1, _N_RAYS)
    dx = rays_d[:, 0].reshape(1, _N_RAYS)
    dy = rays_d[:, 1].reshape(1, _N_RAYS)
    dz = rays_d[:, 2].reshape(1, _N_RAYS)

    idx = _compute_idx(t_mid, ox, oy, oz, dx, dy, dz)

    table = jnp.concatenate([occ_grid, jnp.zeros((_PADN,), f32)])
    occ = _sc_gather(table, idx.reshape(-1))

    nrows = _N_RAYS // 128
    occ3 = occ.reshape(_NSAMP, nrows, 128)
    cx, cy, cz = _composite(
        t_mid, dt, occ3,
        ox.reshape(nrows, 128), oy.reshape(nrows, 128), oz.reshape(nrows, 128),
        dx.reshape(nrows, 128), dy.reshape(nrows, 128), dz.reshape(nrows, 128),
    )
    return jnp.stack(
        [cx.reshape(-1), cy.reshape(-1), cz.reshape(-1)], axis=-1)


# speedup vs baseline: 297.1689x; 1.0429x over previous
import functools

import jax
import jax.numpy as jnp
from jax import lax
from jax.experimental import pallas as pl
from jax.experimental.pallas import tpu as pltpu
from jax.experimental.pallas import tpu_sc as plsc

_N_RAYS = 65536
_RES = 128
_NSAMP = 128
_NEAR = 0.1
_FAR = 3.0
_OCC_THRES = 0.01
_TAB = _RES * _RES * _RES
_PADN = 65536

_RB = 1024
_ROWS = 8


def _idx_body(t_ref, ox_ref, oy_ref, oz_ref, dx_ref, dy_ref, dz_ref, idx_ref):
    b = pl.program_id(0)
    t = t_ref[...]
    dxv, dyv, dzv = dx_ref[...], dy_ref[...], dz_ref[...]
    norm = jnp.sqrt(dxv * dxv + dyv * dyv + dzv * dzv) + 1e-8
    inv = 64.0 / norm
    ax = ox_ref[...] * 64.0 + 64.0
    ay = oy_ref[...] * 64.0 + 64.0
    az = oz_ref[...] * 64.0 + 64.0
    bx, by, bz = dxv * inv, dyv * inv, dzv * inv

    ux = ax + bx * t
    uy = ay + by * t
    uz = az + bz * t
    ix = jnp.floor(ux).astype(jnp.int32)
    iy = jnp.floor(uy).astype(jnp.int32)
    iz = jnp.floor(uz).astype(jnp.int32)
    ior = ix | iy | iz
    inside = ior.astype(jnp.uint32) < jnp.uint32(_RES)
    flat = (ix << 14) + (iy << 7) + iz
    r = lax.broadcasted_iota(jnp.int32, (_NSAMP, _RB), 1) + b * _RB
    s = lax.broadcasted_iota(jnp.int32, (_NSAMP, _RB), 0)
    pad = _TAB + (((r << 7) + s) & (_PADN - 1))
    idx_ref[...] = jnp.where(inside, flat, pad)


def _compute_idx(t_mid, ox, oy, oz, dx, dy, dz):
    nblk = _N_RAYS // _RB
    ray_spec = pl.BlockSpec((1, _RB), lambda b: (0, b))
    return pl.pallas_call(
        _idx_body,
        grid=(nblk,),
        in_specs=[pl.BlockSpec((_NSAMP, 1), lambda b: (0, 0))] + [ray_spec] * 6,
        out_specs=pl.BlockSpec((_NSAMP, _RB), lambda b: (0, b)),
        out_shape=jax.ShapeDtypeStruct((_NSAMP, _N_RAYS), jnp.int32),
    )(t_mid.reshape(_NSAMP, 1), ox, oy, oz, dx, dy, dz)


def _sc_gather(table, idx_flat):
    nw = 32
    b_total = idx_flat.shape[0]
    b_per_w = b_total // nw
    ch = 16384
    nch = b_per_w // ch
    mesh = plsc.VectorSubcoreMesh(core_axis_name="c", subcore_axis_name="s")

    @functools.partial(
        pl.kernel,
        out_type=jax.ShapeDtypeStruct((b_total,), jnp.float32),
        mesh=mesh,
        scratch_types=[
            pltpu.VMEM((ch,), jnp.int32),
            pltpu.VMEM((ch,), jnp.float32),
            pltpu.SemaphoreType.DMA,
        ],
    )
    def gather_k(tab_hbm, idx_hbm, out_hbm, idx_v, occ_v, sem):
        wid = lax.axis_index("s") * 2 + lax.axis_index("c")
        base = wid * b_per_w

        def body(c, carry):
            off = base + c * ch
            pltpu.sync_copy(idx_hbm.at[pl.ds(off, ch)], idx_v)
            pltpu.async_copy(tab_hbm.at[idx_v], occ_v, sem).wait()
            pltpu.sync_copy(occ_v, out_hbm.at[pl.ds(off, ch)])
            return carry

        lax.fori_loop(0, nch, body, 0)

    return gather_k(table, idx_flat)


def _comp_body(t_ref, dt_ref, occ_ref, ox_ref, oy_ref, oz_ref,
               dx_ref, dy_ref, dz_ref, cx_ref, cy_ref, cz_ref):
    dxv, dyv, dzv = dx_ref[...], dy_ref[...], dz_ref[...]
    norm = jnp.sqrt(dxv * dxv + dyv * dyv + dzv * dzv) + 1e-8
    inv = 1.0 / norm
    ndx, ndy, ndz = dxv * inv, dyv * inv, dzv * inv

    shape = (_ROWS, 128)

    def body(sidx, carry):
        cum, e_cum, wsum, tsum = carry
        occ_s = occ_ref[sidx]
        sp = jnp.log1p(jnp.exp(occ_s))
        sigma = jnp.where(occ_s > _OCC_THRES, sp, 0.0)
        new_cum = cum - sigma * dt_ref[sidx]
        e_new = jnp.exp(new_cum)
        w = e_cum - e_new
        wsum = wsum + w
        tsum = tsum + w * t_ref[sidx]
        return new_cum, e_new, wsum, tsum

    init = (jnp.zeros(shape, jnp.float32),
            jnp.ones(shape, jnp.float32),
            jnp.zeros(shape, jnp.float32),
            jnp.zeros(shape, jnp.float32))
    _, _, wsum, tsum = lax.fori_loop(0, _NSAMP, body, init)

    cx_ref[...] = ox_ref[...] * wsum + ndx * tsum
    cy_ref[...] = oy_ref[...] * wsum + ndy * tsum
    cz_ref[...] = oz_ref[...] * wsum + ndz * tsum


def _composite(t_mid, dt, occ, ox, oy, oz, dx, dy, dz):
    nrows = _N_RAYS // 128
    nblk = nrows // _ROWS
    smem_spec = pl.BlockSpec(memory_space=pltpu.SMEM)
    ray_spec = pl.BlockSpec((_ROWS, 128), lambda b: (b, 0))
    out_sds = jax.ShapeDtypeStruct((nrows, 128), jnp.float32)
    return pl.pallas_call(
        _comp_body,
        grid=(nblk,),
        in_specs=[smem_spec, smem_spec,
                  pl.BlockSpec((_NSAMP, _ROWS, 128), lambda b: (0, b, 0))]
                 + [ray_spec] * 6,
        out_specs=[ray_spec] * 3,
        out_shape=[out_sds, out_sds, out_sds],
    )(t_mid, dt, occ, ox, oy, oz, dx, dy, dz)


def kernel(rays_o, rays_d, occ_grid):
    f32 = jnp.float32
    t_edges = jnp.linspace(_NEAR, _FAR, _NSAMP + 1, dtype=f32)
    t_mid = 0.5 * (t_edges[:-1] + t_edges[1:])
    dt = t_edges[1:] - t_edges[:-1]

    ox = rays_o[:, 0].reshape(1, _N_RAYS)
    oy = rays_o[:, 1].reshape(1, _N_RAYS)
    oz = rays_o[:, 2].reshape(1, _N_RAYS)
    dx = rays_d[:, 0].reshape(1, _N_RAYS)
    dy = rays_d[:, 1].reshape(1, _N_RAYS)
    dz = rays_d[:, 2].reshape(1, _N_RAYS)

    idx = _compute_idx(t_mid, ox, oy, oz, dx, dy, dz)

    table = jnp.concatenate([occ_grid, jnp.zeros((_PADN,), f32)])
    occ = _sc_gather(table, idx.reshape(-1))

    nrows = _N_RAYS // 128
    occ3 = occ.reshape(_NSAMP, nrows, 128)
    cx, cy, cz = _composite(
        t_mid, dt, occ3,
        ox.reshape(nrows, 128), oy.reshape(nrows, 128), oz.reshape(nrows, 128),
        dx.reshape(nrows, 128), dy.reshape(nrows, 128), dz.reshape(nrows, 128),
    )
    return jnp.stack(
        [cx.reshape(-1), cy.reshape(-1), cz.reshape(-1)], axis=-1)


# speedup vs baseline: 348.0113x; 1.1711x over previous
import functools

import jax
import jax.numpy as jnp
from jax import lax
from jax.experimental import pallas as pl
from jax.experimental.pallas import tpu as pltpu
from jax.experimental.pallas import tpu_sc as plsc

_N_RAYS = 65536
_RES = 128
_NSAMP = 128
_NEAR = 0.1
_FAR = 3.0
_OCC_THRES = 0.01
_TAB = _RES * _RES * _RES
_PADN = 65536

_RB = 1024
_ROWS = 8


def _idx_body(t_ref, ox_ref, oy_ref, oz_ref, dx_ref, dy_ref, dz_ref, idx_ref):
    b = pl.program_id(0)
    t = t_ref[...]
    dxv, dyv, dzv = dx_ref[...], dy_ref[...], dz_ref[...]
    norm = jnp.sqrt(dxv * dxv + dyv * dyv + dzv * dzv) + 1e-8
    inv = 64.0 / norm
    ax = ox_ref[...] * 64.0 + 64.0
    ay = oy_ref[...] * 64.0 + 64.0
    az = oz_ref[...] * 64.0 + 64.0
    bx, by, bz = dxv * inv, dyv * inv, dzv * inv

    ux = ax + bx * t
    uy = ay + by * t
    uz = az + bz * t
    ix = jnp.floor(ux).astype(jnp.int32)
    iy = jnp.floor(uy).astype(jnp.int32)
    iz = jnp.floor(uz).astype(jnp.int32)
    ior = ix | iy | iz
    inside = ior.astype(jnp.uint32) < jnp.uint32(_RES)
    flat = (ix << 14) + (iy << 7) + iz
    r = lax.broadcasted_iota(jnp.int32, (_NSAMP, _RB), 1) + b * _RB
    s = lax.broadcasted_iota(jnp.int32, (_NSAMP, _RB), 0)
    pad = _TAB + (((r << 7) + s) & (_PADN - 1))
    idx_ref[...] = jnp.where(inside, flat, pad)


def _compute_idx(t_mid, ox, oy, oz, dx, dy, dz):
    nblk = _N_RAYS // _RB
    ray_spec = pl.BlockSpec((1, _RB), lambda b: (0, b))
    return pl.pallas_call(
        _idx_body,
        grid=(nblk,),
        in_specs=[pl.BlockSpec((_NSAMP, 1), lambda b: (0, 0))] + [ray_spec] * 6,
        out_specs=pl.BlockSpec((_NSAMP, _RB), lambda b: (0, b)),
        out_shape=jax.ShapeDtypeStruct((_NSAMP, _N_RAYS), jnp.int32),
    )(t_mid.reshape(_NSAMP, 1), ox, oy, oz, dx, dy, dz)


def _sc_gather(table, idx_flat):
    nw = 32
    b_total = idx_flat.shape[0]
    b_per_w = b_total // nw
    ch = 16384
    nch = b_per_w // ch
    mesh = plsc.VectorSubcoreMesh(core_axis_name="c", subcore_axis_name="s")

    nbuf = 2
    nsub = 4
    sub = ch // nsub
    nrows_w = b_per_w // sub

    vmem_i = [pltpu.VMEM((sub,), jnp.int32) for _ in range(nbuf * nsub)]
    vmem_f = [pltpu.VMEM((sub,), jnp.float32) for _ in range(nbuf * nsub)]

    @functools.partial(
        pl.kernel,
        out_type=jax.ShapeDtypeStruct((b_total // sub, sub), jnp.float32),
        mesh=mesh,
        scratch_types=vmem_i + vmem_f + [
            pltpu.SemaphoreType.DMA,
            pltpu.SemaphoreType.DMA,
            pltpu.SemaphoreType.DMA,
        ],
    )
    def gather_k(tab_hbm, idx_hbm, out_hbm, *refs):
        idx_v = [list(refs[b * nsub:(b + 1) * nsub]) for b in range(nbuf)]
        occ_v = [list(refs[nbuf * nsub + b * nsub:
                           nbuf * nsub + (b + 1) * nsub])
                 for b in range(nbuf)]
        sem_in, sem_g, sem_out = refs[2 * nbuf * nsub:]
        wid = lax.axis_index("s") * 2 + lax.axis_index("c")
        row0 = wid * nrows_w

        def stage_in(c):
            b = c % nbuf
            return [pltpu.async_copy(
                        idx_hbm.at[row0 + c * nsub + i], idx_v[b][i], sem_in)
                    for i in range(nsub)]

        in_descs = [None] * nch
        out_descs = [None] * nch
        in_descs[0] = stage_in(0)
        for c in range(nch):
            b = c % nbuf
            for d in in_descs[c]:
                d.wait()
            if c + 1 < nch:
                in_descs[c + 1] = stage_in(c + 1)
            if c >= nbuf:
                for d in out_descs[c - nbuf]:
                    d.wait()
            gd = [pltpu.async_copy(tab_hbm.at[idx_v[b][i]], occ_v[b][i],
                                   sem_g)
                  for i in range(nsub)]
            for d in gd:
                d.wait()
            out_descs[c] = [
                pltpu.async_copy(occ_v[b][i],
                                 out_hbm.at[row0 + c * nsub + i], sem_out)
                for i in range(nsub)
            ]
        for c in range(nch - nbuf, nch):
            for d in out_descs[c]:
                d.wait()

    return gather_k(table, idx_flat.reshape(b_total // sub, sub)).reshape(
        b_total)


def _comp_body(t_ref, dt_ref, occ_ref, ox_ref, oy_ref, oz_ref,
               dx_ref, dy_ref, dz_ref, cx_ref, cy_ref, cz_ref):
    dxv, dyv, dzv = dx_ref[...], dy_ref[...], dz_ref[...]
    norm = jnp.sqrt(dxv * dxv + dyv * dyv + dzv * dzv) + 1e-8
    inv = 1.0 / norm
    ndx, ndy, ndz = dxv * inv, dyv * inv, dzv * inv

    occ = occ_ref[...]
    sp = jnp.log1p(jnp.exp(occ))
    sigma = jnp.where(occ > _OCC_THRES, sp, 0.0)
    cum = sigma * dt_ref[...].reshape(_NSAMP, 1, 1)
    k = 1
    while k < _NSAMP:
        z = jnp.zeros((k, _ROWS, 128), jnp.float32)
        cum = cum + jnp.concatenate([z, cum[:-k]], axis=0)
        k *= 2
    e = jnp.exp(-cum)
    e_prev = jnp.concatenate(
        [jnp.ones((1, _ROWS, 128), jnp.float32), e[:-1]], axis=0)
    w = e_prev - e
    wsum = 1.0 - e[_NSAMP - 1]
    tsum = jnp.sum(w * t_ref[...].reshape(_NSAMP, 1, 1), axis=0)

    cx_ref[...] = ox_ref[...] * wsum + ndx * tsum
    cy_ref[...] = oy_ref[...] * wsum + ndy * tsum
    cz_ref[...] = oz_ref[...] * wsum + ndz * tsum


def _composite(t_mid, dt, occ, ox, oy, oz, dx, dy, dz):
    nrows = _N_RAYS // 128
    nblk = nrows // _ROWS
    tcol_spec = pl.BlockSpec((_NSAMP, 1), lambda b: (0, 0))
    ray_spec = pl.BlockSpec((_ROWS, 128), lambda b: (b, 0))
    out_sds = jax.ShapeDtypeStruct((nrows, 128), jnp.float32)
    return pl.pallas_call(
        _comp_body,
        grid=(nblk,),
        in_specs=[tcol_spec, tcol_spec,
                  pl.BlockSpec((_NSAMP, _ROWS, 128), lambda b: (0, b, 0))]
                 + [ray_spec] * 6,
        out_specs=[ray_spec] * 3,
        out_shape=[out_sds, out_sds, out_sds],
    )(t_mid.reshape(_NSAMP, 1), dt.reshape(_NSAMP, 1),
      occ, ox, oy, oz, dx, dy, dz)


def kernel(rays_o, rays_d, occ_grid):
    f32 = jnp.float32
    t_edges = jnp.linspace(_NEAR, _FAR, _NSAMP + 1, dtype=f32)
    t_mid = 0.5 * (t_edges[:-1] + t_edges[1:])
    dt = t_edges[1:] - t_edges[:-1]

    ox = rays_o[:, 0].reshape(1, _N_RAYS)
    oy = rays_o[:, 1].reshape(1, _N_RAYS)
    oz = rays_o[:, 2].reshape(1, _N_RAYS)
    dx = rays_d[:, 0].reshape(1, _N_RAYS)
    dy = rays_d[:, 1].reshape(1, _N_RAYS)
    dz = rays_d[:, 2].reshape(1, _N_RAYS)

    idx = _compute_idx(t_mid, ox, oy, oz, dx, dy, dz)

    table = jnp.concatenate([occ_grid, jnp.zeros((_PADN,), f32)])
    occ = _sc_gather(table, idx.reshape(-1))

    nrows = _N_RAYS // 128
    occ3 = occ.reshape(_NSAMP, nrows, 128)
    cx, cy, cz = _composite(
        t_mid, dt, occ3,
        ox.reshape(nrows, 128), oy.reshape(nrows, 128), oz.reshape(nrows, 128),
        dx.reshape(nrows, 128), dy.reshape(nrows, 128), dz.reshape(nrows, 128),
    )
    return jnp.stack(
        [cx.reshape(-1), cy.reshape(-1), cz.reshape(-1)], axis=-1)


# speedup vs baseline: 482.1568x; 1.3855x over previous
import functools

import jax
import jax.numpy as jnp
from jax import lax
from jax.experimental import pallas as pl
from jax.experimental.pallas import tpu as pltpu
from jax.experimental.pallas import tpu_sc as plsc

_N_RAYS = 65536
_RES = 128
_NSAMP = 128
_NEAR = 0.1
_FAR = 3.0
_OCC_THRES = 0.01
_TAB = _RES * _RES * _RES

_SBLK = 8
_RV = _N_RAYS // 1024


def _idx_body(t_ref, ox_ref, oy_ref, oz_ref, dx_ref, dy_ref, dz_ref, idx_ref):
    j = pl.program_id(0)
    dxv, dyv, dzv = dx_ref[...], dy_ref[...], dz_ref[...]
    norm = jnp.sqrt(dxv * dxv + dyv * dyv + dzv * dzv) + 1e-8
    inv = 64.0 / norm
    ax = ox_ref[...] * 64.0 + 64.0
    ay = oy_ref[...] * 64.0 + 64.0
    az = oz_ref[...] * 64.0 + 64.0
    bx, by, bz = dxv * inv, dyv * inv, dzv * inv

    for sl in range(_SBLK):
        ts = t_ref[j * _SBLK + sl]
        ix = jnp.floor(ax + bx * ts).astype(jnp.int32)
        iy = jnp.floor(ay + by * ts).astype(jnp.int32)
        iz = jnp.floor(az + bz * ts).astype(jnp.int32)
        flat = ((ix << 14) + (iy << 7) + iz) & (_TAB - 1)
        idx_ref[pl.ds(sl * _RV, _RV)] = flat


def _compute_idx(t_mid, ox, oy, oz, dx, dy, dz):
    nblk = _NSAMP // _SBLK
    ray_spec = pl.BlockSpec((_RV, 8, 128), lambda b: (0, 0, 0))
    return pl.pallas_call(
        _idx_body,
        grid=(nblk,),
        in_specs=[pl.BlockSpec(memory_space=pltpu.SMEM)] + [ray_spec] * 6,
        out_specs=pl.BlockSpec((_SBLK * _RV, 8, 128), lambda b: (b, 0, 0)),
        out_shape=jax.ShapeDtypeStruct((_NSAMP * _RV, 8, 128), jnp.int32),
    )(t_mid, ox, oy, oz, dx, dy, dz)


def _sc_gather(table, idx_flat):
    nw = 32
    b_total = idx_flat.shape[0]
    b_per_w = b_total // nw
    ch = 16384
    nch = b_per_w // ch
    nbuf = 2
    mesh = plsc.VectorSubcoreMesh(core_axis_name="c", subcore_axis_name="s")

    @functools.partial(
        pl.kernel,
        out_type=jax.ShapeDtypeStruct((b_total,), jnp.float32),
        mesh=mesh,
        scratch_types=[
            pltpu.VMEM((ch,), jnp.int32),
            pltpu.VMEM((ch,), jnp.int32),
            pltpu.VMEM((ch,), jnp.float32),
            pltpu.VMEM((ch,), jnp.float32),
            pltpu.SemaphoreType.DMA,
            pltpu.SemaphoreType.DMA,
            pltpu.SemaphoreType.DMA,
        ],
    )
    def gather_k(tab_hbm, idx_hbm, out_hbm, idx_v0, idx_v1, occ_v0, occ_v1,
                 sem_in, sem_g, sem_out):
        idx_v = [idx_v0, idx_v1]
        occ_v = [occ_v0, occ_v1]
        wid = lax.axis_index("s") * 2 + lax.axis_index("c")
        base = wid * b_per_w

        def stage_in(c):
            return pltpu.async_copy(
                idx_hbm.at[pl.ds(base + c * ch, ch)], idx_v[c % nbuf],
                sem_in)

        in_descs = [None] * nch
        out_descs = [None] * nch
        in_descs[0] = stage_in(0)
        for c in range(nch):
            b = c % nbuf
            in_descs[c].wait()
            if c + 1 < nch:
                in_descs[c + 1] = stage_in(c + 1)
            if c >= nbuf:
                out_descs[c - nbuf].wait()
            pltpu.async_copy(tab_hbm.at[idx_v[b]], occ_v[b], sem_g).wait()
            out_descs[c] = pltpu.async_copy(
                occ_v[b], out_hbm.at[pl.ds(base + c * ch, ch)], sem_out)
        for c in range(nch - nbuf, nch):
            out_descs[c].wait()

    return gather_k(table, idx_flat)


def _comp_body(t_ref, dt_ref, occ_ref, ox_ref, oy_ref, oz_ref,
               dx_ref, dy_ref, dz_ref, cx_ref, cy_ref, cz_ref):
    oxv = ox_ref[...][0]
    oyv = oy_ref[...][0]
    ozv = oz_ref[...][0]
    dxv = dx_ref[...][0]
    dyv = dy_ref[...][0]
    dzv = dz_ref[...][0]
    norm = jnp.sqrt(dxv * dxv + dyv * dyv + dzv * dzv) + 1e-8
    inv = 1.0 / norm
    ndx, ndy, ndz = dxv * inv, dyv * inv, dzv * inv

    def slab(a, b):
        r = 1.0 / (b * 64.0)
        la = (0.0 - a) * r
        lb = (128.0 - a) * r
        return jnp.minimum(la, lb), jnp.maximum(la, lb)

    lox, hix = slab(oxv * 64.0 + 64.0, ndx)
    loy, hiy = slab(oyv * 64.0 + 64.0, ndy)
    loz, hiz = slab(ozv * 64.0 + 64.0, ndz)
    t_lo = jnp.maximum(jnp.maximum(lox, loy), loz)
    t_hi = jnp.minimum(jnp.minimum(hix, hiy), hiz)

    occ = occ_ref[...][:, 0]
    t3 = t_ref[...].reshape(_NSAMP, 1, 1)
    inside = (t3 >= t_lo) & (t3 <= t_hi)
    sp = jnp.log1p(jnp.exp(occ))
    sigma = jnp.where((occ > _OCC_THRES) & inside, sp, 0.0)
    cum = sigma * dt_ref[...].reshape(_NSAMP, 1, 1)
    k = 1
    while k < _NSAMP:
        z = jnp.zeros((k, 8, 128), jnp.float32)
        cum = cum + jnp.concatenate([z, cum[:-k]], axis=0)
        k *= 2
    e = jnp.exp(-cum)
    e_prev = jnp.concatenate(
        [jnp.ones((1, 8, 128), jnp.float32), e[:-1]], axis=0)
    w = e_prev - e
    wsum = 1.0 - e[_NSAMP - 1]
    tsum = jnp.sum(w * t3, axis=0)

    cx_ref[0] = oxv * wsum + ndx * tsum
    cy_ref[0] = oyv * wsum + ndy * tsum
    cz_ref[0] = ozv * wsum + ndz * tsum


def _composite(t_mid, dt, occ4, ox, oy, oz, dx, dy, dz):
    tcol_spec = pl.BlockSpec((_NSAMP, 1), lambda b: (0, 0))
    ray_spec = pl.BlockSpec((1, 8, 128), lambda b: (b, 0, 0))
    out_sds = jax.ShapeDtypeStruct((_RV, 8, 128), jnp.float32)
    return pl.pallas_call(
        _comp_body,
        grid=(_RV,),
        in_specs=[tcol_spec, tcol_spec,
                  pl.BlockSpec((_NSAMP, 1, 8, 128), lambda b: (0, b, 0, 0))]
                 + [ray_spec] * 6,
        out_specs=[ray_spec] * 3,
        out_shape=[out_sds, out_sds, out_sds],
    )(t_mid.reshape(_NSAMP, 1), dt.reshape(_NSAMP, 1),
      occ4, ox, oy, oz, dx, dy, dz)


def kernel(rays_o, rays_d, occ_grid):
    f32 = jnp.float32
    t_edges = jnp.linspace(_NEAR, _FAR, _NSAMP + 1, dtype=f32)
    t_mid = 0.5 * (t_edges[:-1] + t_edges[1:])
    dt = t_edges[1:] - t_edges[:-1]

    ox = rays_o[:, 0].reshape(_RV, 8, 128)
    oy = rays_o[:, 1].reshape(_RV, 8, 128)
    oz = rays_o[:, 2].reshape(_RV, 8, 128)
    dx = rays_d[:, 0].reshape(_RV, 8, 128)
    dy = rays_d[:, 1].reshape(_RV, 8, 128)
    dz = rays_d[:, 2].reshape(_RV, 8, 128)

    idx3 = _compute_idx(t_mid, ox, oy, oz, dx, dy, dz)
    occ = _sc_gather(occ_grid, idx3.reshape(-1))
    occ4 = occ.reshape(_NSAMP, _RV, 8, 128)
    cx, cy, cz = _composite(t_mid, dt, occ4, ox, oy, oz, dx, dy, dz)
    return jnp.stack(
        [cx.reshape(-1), cy.reshape(-1), cz.reshape(-1)], axis=-1)
